# baseline (speedup 1.0000x reference)
import numpy as np
import jax
import jax.numpy as jnp
from jax.experimental import pallas as pl
from jax.experimental.pallas import tpu as pltpu

_CH = 256


def _chunks(m):
    o = 0
    while o < m:
        l = min(_CH, m - o)
        yield o, l
        o += l


def _eye(k):
    r = jax.lax.broadcasted_iota(jnp.int32, (k, k), 0)
    c = jax.lax.broadcasted_iota(jnp.int32, (k, k), 1)
    return jnp.where(r == c, 1.0, 0.0).astype(jnp.float32)


def _mxu_t(x):
    a = x.shape[0]
    return jax.lax.dot_general(x, _eye(a), (((0,), (0,)), ((), ())),
                               preferred_element_type=jnp.float32)


def _tree_dp_kernel(feat_ref, emb_ref, out_ref,
                    a_ref, nrm_ref, embt_ref, ewl_ref, ewr_ref):
    c, n = feat_ref.shape
    ce = emb_ref.shape[0]
    K = int(np.log2(n))
    TC = min(512, n)

    for k in range(n // TC):
        cs = slice(k * TC, (k + 1) * TC)
        a_ref[cs, :] = _mxu_t(feat_ref[:, cs])
        embt_ref[cs, :] = _mxu_t(emb_ref[:, cs])

    for o, l in _chunks(n // 2):
        nrm_ref[n // 2 + o:n // 2 + o + l, :] = jnp.ones((l, c), jnp.float32)

    pr = n // 2 - 1
    dl = embt_ref[n - 1:n, :] - embt_ref[pr:pr + 1, :]
    wl = jnp.broadcast_to(jnp.exp(-jnp.sum(dl * dl, axis=1, keepdims=True)), (1, c))
    ewl_ref[pr:pr + 1, :] = wl
    a_ref[pr:pr + 1, :] += wl * a_ref[n - 1:n, :]
    nrm_ref[pr:pr + 1, :] = 1.0 + wl * nrm_ref[n - 1:n, :]

    for d in range(K - 1, 0, -1):
        s = 2**d - 1
        sp, m2 = 2 ** (d - 1) - 1, 2 ** (d - 1)
        for o, l in _chunks(m2):
            rp = slice(sp + o, sp + o + l)
            rl = slice(s + 2 * o, s + 2 * o + 2 * l, 2)
            rr = slice(s + 2 * o + 1, s + 2 * o + 2 * l, 2)
            ep = embt_ref[rp, :]
            dl = embt_ref[rl, :] - ep
            dr = embt_ref[rr, :] - ep
            wl = jnp.broadcast_to(
                jnp.exp(-jnp.sum(dl * dl, axis=1, keepdims=True)), (l, c))
            wr = jnp.broadcast_to(
                jnp.exp(-jnp.sum(dr * dr, axis=1, keepdims=True)), (l, c))
            ewl_ref[rp, :] = wl
            ewr_ref[rp, :] = wr
            a_ref[rp, :] += wl * a_ref[rl, :] + wr * a_ref[rr, :]
            nrm_ref[rp, :] = 1.0 + wl * nrm_ref[rl, :] + wr * nrm_ref[rr, :]

    for d in range(1, K):
        s = 2**d - 1
        sp, m2 = 2 ** (d - 1) - 1, 2 ** (d - 1)
        for o, l in _chunks(m2):
            rp = slice(sp + o, sp + o + l)
            rl = slice(s + 2 * o, s + 2 * o + 2 * l, 2)
            rr = slice(s + 2 * o + 1, s + 2 * o + 2 * l, 2)
            wl = ewl_ref[rp, :]
            wr = ewr_ref[rp, :]
            p = a_ref[rp, :]
            pn = nrm_ref[rp, :]
            al = a_ref[rl, :]
            ar = a_ref[rr, :]
            a_ref[rl, :] = al + wl * (p - wl * al)
            a_ref[rr, :] = ar + wr * (p - wr * ar)
            nl = nrm_ref[rl, :]
            nr = nrm_ref[rr, :]
            nrm_ref[rl, :] = nl + wl * (pn - wl * nl)
            nrm_ref[rr, :] = nr + wr * (pn - wr * nr)
    wl = ewl_ref[pr:pr + 1, :]
    a = a_ref[n - 1:n, :]
    a_ref[n - 1:n, :] = a + wl * (a_ref[pr:pr + 1, :] - wl * a)
    nn = nrm_ref[n - 1:n, :]
    nrm_ref[n - 1:n, :] = nn + wl * (nrm_ref[pr:pr + 1, :] - wl * nn)

    for k in range(n // TC):
        cs = slice(k * TC, (k + 1) * TC)
        out_ref[:, cs] = _mxu_t(a_ref[cs, :] / nrm_ref[cs, :])


def _run(feat, emb):
    c, n = feat.shape
    ce = emb.shape[0]
    return pl.pallas_call(
        _tree_dp_kernel,
        out_shape=jax.ShapeDtypeStruct((c, n), jnp.float32),
        scratch_shapes=[
            pltpu.VMEM((n, c), jnp.float32),
            pltpu.VMEM((n, c), jnp.float32),
            pltpu.VMEM((n, ce), jnp.float32),
            pltpu.VMEM((n // 2, c), jnp.float32),
            pltpu.VMEM((n // 2, c), jnp.float32),
        ],
    )(feat, emb)


def kernel(feature_in, embed_in, tree):
    b, c, h, w = feature_in.shape
    n = h * w
    ce = embed_in.shape[1]
    feat = feature_in.reshape(b, c, n)
    emb = embed_in.reshape(b, ce, n)
    out = jnp.stack([_run(feat[i], emb[i]) for i in range(b)])
    return out.reshape(b, c, h, w)


# speedup vs baseline: 1.2716x; 1.2716x over previous
import numpy as np
import jax
import jax.numpy as jnp
from jax.experimental import pallas as pl
from jax.experimental.pallas import tpu as pltpu

_CH = 256


def _chunks(m):
    o = 0
    while o < m:
        l = min(_CH, m - o)
        yield o, l
        o += l


def _eye(k):
    r = jax.lax.broadcasted_iota(jnp.int32, (k, k), 0)
    c = jax.lax.broadcasted_iota(jnp.int32, (k, k), 1)
    return jnp.where(r == c, 1.0, 0.0).astype(jnp.float32)


def _mxu_t(x):
    a = x.shape[0]
    return jax.lax.dot_general(x, _eye(a), (((0,), (0,)), ((), ())),
                               preferred_element_type=jnp.float32)


def _tree_dp_kernel(feat_hbm, emb_hbm, out_hbm,
                    a_ref, nrm_ref, ewl_ref, ewr_ref, es_ref, fs_ref, os_ref,
                    esem, fsem, osem):
    b, c, n = feat_hbm.shape
    ce = emb_hbm.shape[1]
    eh = es_ref.shape[0]
    K = int(np.log2(n))
    TC = min(512, n)
    i = pl.program_id(0)

    @pl.when(i == 0)
    def _():
        pltpu.make_async_copy(feat_hbm.at[0], fs_ref, fsem).start()

    for j in range(ce // eh):
        cp = pltpu.make_async_copy(
            emb_hbm.at[i, pl.ds(j * eh, eh), :], es_ref, esem)
        cp.start()
        cp.wait()
        for k in range(n // TC):
            cs = slice(k * TC, (k + 1) * TC)
            a_ref[cs, j * eh:(j + 1) * eh] = _mxu_t(es_ref[:, cs])

    def _ew(rch, rpar):
        dd = a_ref[rch, :ce] - a_ref[rpar, :ce]
        return jnp.broadcast_to(
            jnp.exp(-jnp.sum(dd * dd, axis=1, keepdims=True)), (dd.shape[0], c))

    pr = n // 2 - 1
    ewl_ref[pr:pr + 1, :] = _ew(slice(n - 1, n), slice(pr, pr + 1))
    for d in range(1, K):
        s = 2**d - 1
        sp, m2 = 2 ** (d - 1) - 1, 2 ** (d - 1)
        for o, l in _chunks(m2):
            rp = slice(sp + o, sp + o + l)
            rl = slice(s + 2 * o, s + 2 * o + 2 * l, 2)
            rr = slice(s + 2 * o + 1, s + 2 * o + 2 * l, 2)
            ewl_ref[rp, :] = _ew(rl, rp)
            ewr_ref[rp, :] = _ew(rr, rp)

    pltpu.make_async_copy(feat_hbm.at[i], fs_ref, fsem).wait()
    for k in range(n // TC):
        cs = slice(k * TC, (k + 1) * TC)
        a_ref[cs, :] = _mxu_t(fs_ref[:, cs])

    @pl.when(i + 1 < b)
    def _():
        pltpu.make_async_copy(feat_hbm.at[i + 1], fs_ref, fsem).start()

    for o, l in _chunks(n // 2):
        nrm_ref[n // 2 + o:n // 2 + o + l, :] = jnp.ones((l, c), jnp.float32)

    wl = ewl_ref[pr:pr + 1, :]
    a_ref[pr:pr + 1, :] += wl * a_ref[n - 1:n, :]
    nrm_ref[pr:pr + 1, :] = 1.0 + wl * nrm_ref[n - 1:n, :]

    for d in range(K - 1, 0, -1):
        s = 2**d - 1
        sp, m2 = 2 ** (d - 1) - 1, 2 ** (d - 1)
        for o, l in _chunks(m2):
            rp = slice(sp + o, sp + o + l)
            rl = slice(s + 2 * o, s + 2 * o + 2 * l, 2)
            rr = slice(s + 2 * o + 1, s + 2 * o + 2 * l, 2)
            wl = ewl_ref[rp, :]
            wr = ewr_ref[rp, :]
            a_ref[rp, :] += wl * a_ref[rl, :] + wr * a_ref[rr, :]
            nrm_ref[rp, :] = 1.0 + wl * nrm_ref[rl, :] + wr * nrm_ref[rr, :]

    for d in range(1, K):
        s = 2**d - 1
        sp, m2 = 2 ** (d - 1) - 1, 2 ** (d - 1)
        for o, l in _chunks(m2):
            rp = slice(sp + o, sp + o + l)
            rl = slice(s + 2 * o, s + 2 * o + 2 * l, 2)
            rr = slice(s + 2 * o + 1, s + 2 * o + 2 * l, 2)
            wl = ewl_ref[rp, :]
            wr = ewr_ref[rp, :]
            p = a_ref[rp, :]
            pn = nrm_ref[rp, :]
            al = a_ref[rl, :]
            ar = a_ref[rr, :]
            a_ref[rl, :] = al + wl * (p - wl * al)
            a_ref[rr, :] = ar + wr * (p - wr * ar)
            nl = nrm_ref[rl, :]
            nr = nrm_ref[rr, :]
            nrm_ref[rl, :] = nl + wl * (pn - wl * nl)
            nrm_ref[rr, :] = nr + wr * (pn - wr * nr)
    wl = ewl_ref[pr:pr + 1, :]
    a = a_ref[n - 1:n, :]
    a_ref[n - 1:n, :] = a + wl * (a_ref[pr:pr + 1, :] - wl * a)
    nn = nrm_ref[n - 1:n, :]
    nrm_ref[n - 1:n, :] = nn + wl * (nrm_ref[pr:pr + 1, :] - wl * nn)

    @pl.when(i > 0)
    def _():
        pltpu.make_async_copy(os_ref, out_hbm.at[i - 1], osem).wait()

    for k in range(n // TC):
        cs = slice(k * TC, (k + 1) * TC)
        os_ref[:, cs] = _mxu_t(a_ref[cs, :] / nrm_ref[cs, :])
    ocp = pltpu.make_async_copy(os_ref, out_hbm.at[i], osem)
    ocp.start()

    @pl.when(i == b - 1)
    def _():
        ocp.wait()


def kernel(feature_in, embed_in, tree):
    b, c, h, w = feature_in.shape
    n = h * w
    ce = embed_in.shape[1]
    out = pl.pallas_call(
        _tree_dp_kernel,
        grid=(b,),
        in_specs=[
            pl.BlockSpec(memory_space=pltpu.MemorySpace.HBM),
            pl.BlockSpec(memory_space=pltpu.MemorySpace.HBM),
        ],
        out_specs=pl.BlockSpec(memory_space=pltpu.MemorySpace.HBM),
        out_shape=jax.ShapeDtypeStruct((b, c, n), jnp.float32),
        scratch_shapes=[
            pltpu.VMEM((n, c), jnp.float32),
            pltpu.VMEM((n, c), jnp.float32),
            pltpu.VMEM((n // 2, c), jnp.float32),
            pltpu.VMEM((n // 2, c), jnp.float32),
            pltpu.VMEM((min(32, ce), n), jnp.float32),
            pltpu.VMEM((c, n), jnp.float32),
            pltpu.VMEM((c, n), jnp.float32),
            pltpu.SemaphoreType.DMA,
            pltpu.SemaphoreType.DMA,
            pltpu.SemaphoreType.DMA,
        ],
        compiler_params=pltpu.CompilerParams(
            dimension_semantics=("arbitrary",)),
    )(feature_in.reshape(b, c, n), embed_in.reshape(b, ce, n))
    return out.reshape(b, c, h, w)


# speedup vs baseline: 2.1118x; 1.6608x over previous
import numpy as np
import jax
import jax.numpy as jnp
from jax.experimental import pallas as pl
from jax.experimental.pallas import tpu as pltpu

_CH = 256


def _chunks(m):
    o = 0
    while o < m:
        l = min(_CH, m - o)
        yield o, l
        o += l


def _eye(k):
    r = jax.lax.broadcasted_iota(jnp.int32, (k, k), 0)
    c = jax.lax.broadcasted_iota(jnp.int32, (k, k), 1)
    return jnp.where(r == c, 1.0, 0.0).astype(jnp.float32)


def _mxu_t(x):
    a = x.shape[0]
    return jax.lax.dot_general(x, _eye(a), (((0,), (0,)), ((), ())),
                               preferred_element_type=jnp.float32)


def _tree_dp_kernel(feat_hbm, emb_hbm, out_hbm,
                    a_ref, nrm_ref, ewl_ref, ewr_ref, es_ref, fs_ref, os_ref,
                    esem, fsem, osem):
    b, c, h, w = feat_hbm.shape
    ce = emb_hbm.shape[1]
    eh = es_ref.shape[0]
    n = h * w
    K = int(np.log2(n))
    i = pl.program_id(0)

    @pl.when(i == 0)
    def _():
        pltpu.make_async_copy(feat_hbm.at[0], fs_ref, fsem).start()

    for j in range(ce // eh):
        cp = pltpu.make_async_copy(
            emb_hbm.at[i, pl.ds(j * eh, eh), :, :], es_ref, esem)
        cp.start()
        cp.wait()
        for k in range(h):
            a_ref[k * w:(k + 1) * w, j * eh:(j + 1) * eh] = _mxu_t(
                es_ref[:, k, :])

    def _ew(rch, rpar):
        dd = a_ref[rch, :ce] - a_ref[rpar, :ce]
        return jnp.broadcast_to(
            jnp.exp(-jnp.sum(dd * dd, axis=1, keepdims=True)), (dd.shape[0], c))

    pr = n // 2 - 1
    ewl_ref[pr:pr + 1, :] = _ew(slice(n - 1, n), slice(pr, pr + 1))
    for d in range(1, K):
        s = 2**d - 1
        sp, m2 = 2 ** (d - 1) - 1, 2 ** (d - 1)
        for o, l in _chunks(m2):
            rp = slice(sp + o, sp + o + l)
            rl = slice(s + 2 * o, s + 2 * o + 2 * l, 2)
            rr = slice(s + 2 * o + 1, s + 2 * o + 2 * l, 2)
            ewl_ref[rp, :] = _ew(rl, rp)
            ewr_ref[rp, :] = _ew(rr, rp)

    pltpu.make_async_copy(feat_hbm.at[i], fs_ref, fsem).wait()
    for k in range(h):
        a_ref[k * w:(k + 1) * w, :] = _mxu_t(fs_ref[:, k, :])

    @pl.when(i + 1 < b)
    def _():
        pltpu.make_async_copy(feat_hbm.at[i + 1], fs_ref, fsem).start()

    for o, l in _chunks(n // 2):
        nrm_ref[n // 2 + o:n // 2 + o + l, :] = jnp.ones((l, c), jnp.float32)

    wl = ewl_ref[pr:pr + 1, :]
    a_ref[pr:pr + 1, :] += wl * a_ref[n - 1:n, :]
    nrm_ref[pr:pr + 1, :] = 1.0 + wl * nrm_ref[n - 1:n, :]

    for d in range(K - 1, 0, -1):
        s = 2**d - 1
        sp, m2 = 2 ** (d - 1) - 1, 2 ** (d - 1)
        for o, l in _chunks(m2):
            rp = slice(sp + o, sp + o + l)
            rl = slice(s + 2 * o, s + 2 * o + 2 * l, 2)
            rr = slice(s + 2 * o + 1, s + 2 * o + 2 * l, 2)
            wl = ewl_ref[rp, :]
            wr = ewr_ref[rp, :]
            a_ref[rp, :] += wl * a_ref[rl, :] + wr * a_ref[rr, :]
            nrm_ref[rp, :] = 1.0 + wl * nrm_ref[rl, :] + wr * nrm_ref[rr, :]

    for d in range(1, K):
        s = 2**d - 1
        sp, m2 = 2 ** (d - 1) - 1, 2 ** (d - 1)
        for o, l in _chunks(m2):
            rp = slice(sp + o, sp + o + l)
            rl = slice(s + 2 * o, s + 2 * o + 2 * l, 2)
            rr = slice(s + 2 * o + 1, s + 2 * o + 2 * l, 2)
            wl = ewl_ref[rp, :]
            wr = ewr_ref[rp, :]
            p = a_ref[rp, :]
            pn = nrm_ref[rp, :]
            al = a_ref[rl, :]
            ar = a_ref[rr, :]
            a_ref[rl, :] = al + wl * (p - wl * al)
            a_ref[rr, :] = ar + wr * (p - wr * ar)
            nl = nrm_ref[rl, :]
            nr = nrm_ref[rr, :]
            nrm_ref[rl, :] = nl + wl * (pn - wl * nl)
            nrm_ref[rr, :] = nr + wr * (pn - wr * nr)
    wl = ewl_ref[pr:pr + 1, :]
    a = a_ref[n - 1:n, :]
    a_ref[n - 1:n, :] = a + wl * (a_ref[pr:pr + 1, :] - wl * a)
    nn = nrm_ref[n - 1:n, :]
    nrm_ref[n - 1:n, :] = nn + wl * (nrm_ref[pr:pr + 1, :] - wl * nn)

    @pl.when(i > 0)
    def _():
        pltpu.make_async_copy(os_ref, out_hbm.at[i - 1], osem).wait()

    for k in range(h):
        r = slice(k * w, (k + 1) * w)
        os_ref[:, k, :] = _mxu_t(a_ref[r, :] / nrm_ref[r, :])
    ocp = pltpu.make_async_copy(os_ref, out_hbm.at[i], osem)
    ocp.start()

    @pl.when(i == b - 1)
    def _():
        ocp.wait()


def kernel(feature_in, embed_in, tree):
    b, c, h, w = feature_in.shape
    n = h * w
    ce = embed_in.shape[1]
    return pl.pallas_call(
        _tree_dp_kernel,
        grid=(b,),
        in_specs=[
            pl.BlockSpec(memory_space=pltpu.MemorySpace.HBM),
            pl.BlockSpec(memory_space=pltpu.MemorySpace.HBM),
        ],
        out_specs=pl.BlockSpec(memory_space=pltpu.MemorySpace.HBM),
        out_shape=jax.ShapeDtypeStruct((b, c, h, w), jnp.float32),
        scratch_shapes=[
            pltpu.VMEM((n, c), jnp.float32),
            pltpu.VMEM((n, c), jnp.float32),
            pltpu.VMEM((n // 2, c), jnp.float32),
            pltpu.VMEM((n // 2, c), jnp.float32),
            pltpu.VMEM((min(32, ce), h, w), jnp.float32),
            pltpu.VMEM((c, h, w), jnp.float32),
            pltpu.VMEM((c, h, w), jnp.float32),
            pltpu.SemaphoreType.DMA,
            pltpu.SemaphoreType.DMA,
            pltpu.SemaphoreType.DMA,
        ],
        compiler_params=pltpu.CompilerParams(
            dimension_semantics=("arbitrary",)),
    )(feature_in, embed_in)


# speedup vs baseline: 2.5327x; 1.1993x over previous
import numpy as np
import jax
import jax.numpy as jnp
from jax.experimental import pallas as pl
from jax.experimental.pallas import tpu as pltpu

_CH = 256


def _chunks(m):
    o = 0
    while o < m:
        l = min(_CH, m - o)
        yield o, l
        o += l


def _eye(k):
    r = jax.lax.broadcasted_iota(jnp.int32, (k, k), 0)
    c = jax.lax.broadcasted_iota(jnp.int32, (k, k), 1)
    return jnp.where(r == c, 1.0, 0.0).astype(jnp.float32)


def _mxu_t(x):
    a = x.shape[0]
    return jax.lax.dot_general(x, _eye(a), (((0,), (0,)), ((), ())),
                               preferred_element_type=jnp.float32)


def _tree_dp_kernel(feat_hbm, emb_hbm, out_hbm,
                    a_ref, nrm_ref, ewl_ref, ewr_ref, es_ref, fs_ref, os_ref,
                    esem, fsem, osem):
    b, c, h, w = feat_hbm.shape
    ce = emb_hbm.shape[1]
    n = h * w
    K = int(np.log2(n))
    i = pl.program_id(0)

    @pl.when(i == 0)
    def _():
        pltpu.make_async_copy(feat_hbm.at[0], fs_ref, fsem).start()

    eh = es_ref.shape[0]
    for j in range(ce // eh):
        cp = pltpu.make_async_copy(
            emb_hbm.at[i, pl.ds(j * eh, eh), :, :], es_ref, esem)
        cp.start()
        cp.wait()
        for k in range(0, h, 8):
            x = es_ref[:, k:k + 8, :]
            t = jax.lax.dot_general(x, _eye(eh), (((0,), (0,)), ((), ())),
                                    preferred_element_type=jnp.float32)
            a_ref[k * w:(k + 8) * w, j * eh:(j + 1) * eh] = t.reshape(8 * w, eh)

    def _ew(rch, rpar):
        dd = a_ref[rch, :ce] - a_ref[rpar, :ce]
        return jnp.broadcast_to(
            jnp.exp(-jnp.sum(dd * dd, axis=1, keepdims=True)), (dd.shape[0], c))

    pr = n // 2 - 1
    ewl_ref[pr:pr + 1, :] = _ew(slice(n - 1, n), slice(pr, pr + 1))
    for d in range(1, K):
        s = 2**d - 1
        sp, m2 = 2 ** (d - 1) - 1, 2 ** (d - 1)
        for o, l in _chunks(m2):
            rp = slice(sp + o, sp + o + l)
            rl = slice(s + 2 * o, s + 2 * o + 2 * l, 2)
            rr = slice(s + 2 * o + 1, s + 2 * o + 2 * l, 2)
            ewl_ref[rp, :] = _ew(rl, rp)
            ewr_ref[rp, :] = _ew(rr, rp)

    pltpu.make_async_copy(feat_hbm.at[i], fs_ref, fsem).wait()
    for k in range(0, h, 8):
        x = fs_ref[:, k:k + 8, :]
        t = jax.lax.dot_general(x, _eye(c), (((0,), (0,)), ((), ())),
                                preferred_element_type=jnp.float32)
        a_ref[k * w:(k + 8) * w, :] = t.reshape(8 * w, c)

    @pl.when(i + 1 < b)
    def _():
        pltpu.make_async_copy(feat_hbm.at[i + 1], fs_ref, fsem).start()

    for o, l in _chunks(n // 2):
        nrm_ref[n // 2 + o:n // 2 + o + l, :] = jnp.ones((l, c), jnp.float32)

    wl = ewl_ref[pr:pr + 1, :]
    a_ref[pr:pr + 1, :] += wl * a_ref[n - 1:n, :]
    nrm_ref[pr:pr + 1, :] = 1.0 + wl * nrm_ref[n - 1:n, :]

    for d in range(K - 1, 0, -1):
        s = 2**d - 1
        sp, m2 = 2 ** (d - 1) - 1, 2 ** (d - 1)
        for o, l in _chunks(m2):
            rp = slice(sp + o, sp + o + l)
            rl = slice(s + 2 * o, s + 2 * o + 2 * l, 2)
            rr = slice(s + 2 * o + 1, s + 2 * o + 2 * l, 2)
            wl = ewl_ref[rp, :]
            wr = ewr_ref[rp, :]
            a_ref[rp, :] += wl * a_ref[rl, :] + wr * a_ref[rr, :]
            nrm_ref[rp, :] = 1.0 + wl * nrm_ref[rl, :] + wr * nrm_ref[rr, :]

    for d in range(1, K):
        s = 2**d - 1
        sp, m2 = 2 ** (d - 1) - 1, 2 ** (d - 1)
        for o, l in _chunks(m2):
            rp = slice(sp + o, sp + o + l)
            rl = slice(s + 2 * o, s + 2 * o + 2 * l, 2)
            rr = slice(s + 2 * o + 1, s + 2 * o + 2 * l, 2)
            wl = ewl_ref[rp, :]
            wr = ewr_ref[rp, :]
            p = a_ref[rp, :]
            pn = nrm_ref[rp, :]
            al = a_ref[rl, :]
            ar = a_ref[rr, :]
            a_ref[rl, :] = al + wl * (p - wl * al)
            a_ref[rr, :] = ar + wr * (p - wr * ar)
            nl = nrm_ref[rl, :]
            nr = nrm_ref[rr, :]
            nrm_ref[rl, :] = nl + wl * (pn - wl * nl)
            nrm_ref[rr, :] = nr + wr * (pn - wr * nr)
    wl = ewl_ref[pr:pr + 1, :]
    a = a_ref[n - 1:n, :]
    a_ref[n - 1:n, :] = a + wl * (a_ref[pr:pr + 1, :] - wl * a)
    nn = nrm_ref[n - 1:n, :]
    nrm_ref[n - 1:n, :] = nn + wl * (nrm_ref[pr:pr + 1, :] - wl * nn)

    @pl.when(i > 0)
    def _():
        pltpu.make_async_copy(os_ref, out_hbm.at[i - 1], osem).wait()

    for k in range(0, h, 8):
        r = slice(k * w, (k + 8) * w)
        y = (a_ref[r, :] / nrm_ref[r, :]).reshape(8, w, c)
        os_ref[:, k:k + 8, :] = jax.lax.dot_general(
            _eye(c), y, (((0,), (2,)), ((), ())),
            preferred_element_type=jnp.float32)
    ocp = pltpu.make_async_copy(os_ref, out_hbm.at[i], osem)
    ocp.start()

    @pl.when(i == b - 1)
    def _():
        ocp.wait()


def kernel(feature_in, embed_in, tree):
    b, c, h, w = feature_in.shape
    n = h * w
    ce = embed_in.shape[1]
    return pl.pallas_call(
        _tree_dp_kernel,
        grid=(b,),
        in_specs=[
            pl.BlockSpec(memory_space=pltpu.MemorySpace.HBM),
            pl.BlockSpec(memory_space=pltpu.MemorySpace.HBM),
        ],
        out_specs=pl.BlockSpec(memory_space=pltpu.MemorySpace.HBM),
        out_shape=jax.ShapeDtypeStruct((b, c, h, w), jnp.float32),
        scratch_shapes=[
            pltpu.VMEM((n, c), jnp.float32),
            pltpu.VMEM((n, c), jnp.float32),
            pltpu.VMEM((n // 2, c), jnp.float32),
            pltpu.VMEM((n // 2, c), jnp.float32),
            pltpu.VMEM((min(32, ce), h, w), jnp.float32),
            pltpu.VMEM((c, h, w), jnp.float32),
            pltpu.VMEM((c, h, w), jnp.float32),
            pltpu.SemaphoreType.DMA,
            pltpu.SemaphoreType.DMA,
            pltpu.SemaphoreType.DMA,
        ],
        compiler_params=pltpu.CompilerParams(
            dimension_semantics=("arbitrary",)),
    )(feature_in, embed_in)


# speedup vs baseline: 3.2001x; 1.2635x over previous
import numpy as np
import jax
import jax.numpy as jnp
from jax.experimental import pallas as pl
from jax.experimental.pallas import tpu as pltpu

_CH = 256


def _chunks(m):
    o = 0
    while o < m:
        l = min(_CH, m - o)
        yield o, l
        o += l


def _eye(k):
    r = jax.lax.broadcasted_iota(jnp.int32, (k, k), 0)
    c = jax.lax.broadcasted_iota(jnp.int32, (k, k), 1)
    return jnp.where(r == c, 1.0, 0.0).astype(jnp.float32)


def _t_in(x, k):
    t = jax.lax.dot_general(x, _eye(k), (((0,), (0,)), ((), ())),
                            preferred_element_type=jnp.float32)
    return t.reshape(8 * x.shape[2], k)


def _tree_dp_kernel(feat_hbm, emb_hbm, out_hbm,
                    a_ref, nrm_ref, ewl_ref, ewr_ref, es_ref, fs_ref,
                    osa_ref, osb_ref, esem, fsem, osema, osemb):
    b, c, h, w = feat_hbm.shape
    ce = emb_hbm.shape[1]
    n = h * w
    hq = max(h // 4, 8)
    nq = h // hq
    K = int(np.log2(n))
    i = pl.program_id(0)

    ecp = pltpu.make_async_copy(emb_hbm.at[i], es_ref, esem)
    ecp.start()

    @pl.when(i == 0)
    def _():
        pltpu.make_async_copy(feat_hbm.at[0], fs_ref, fsem).start()

    ecp.wait()
    for k in range(0, h, 8):
        a_ref[k * w:(k + 8) * w, :ce] = _t_in(es_ref[:, k:k + 8, :], ce)

    def _ew(rch, rpar):
        dd = a_ref[rch, :ce] - a_ref[rpar, :ce]
        return jnp.broadcast_to(
            jnp.exp(-jnp.sum(dd * dd, axis=1, keepdims=True)), (dd.shape[0], c))

    pr = n // 2 - 1
    ewl_ref[pr:pr + 1, :] = _ew(slice(n - 1, n), slice(pr, pr + 1))
    for d in range(1, K):
        s = 2**d - 1
        sp, m2 = 2 ** (d - 1) - 1, 2 ** (d - 1)
        for o, l in _chunks(m2):
            rp = slice(sp + o, sp + o + l)
            rl = slice(s + 2 * o, s + 2 * o + 2 * l, 2)
            rr = slice(s + 2 * o + 1, s + 2 * o + 2 * l, 2)
            ewl_ref[rp, :] = _ew(rl, rp)
            ewr_ref[rp, :] = _ew(rr, rp)

    pltpu.make_async_copy(feat_hbm.at[i], fs_ref, fsem).wait()
    for k in range(0, h, 8):
        a_ref[k * w:(k + 8) * w, :] = _t_in(fs_ref[:, k:k + 8, :], c)

    @pl.when(i + 1 < b)
    def _():
        pltpu.make_async_copy(feat_hbm.at[i + 1], fs_ref, fsem).start()

    hw = c // 2
    for o, l in _chunks(n // 2):
        nrm_ref[n // 2 + o:n // 2 + o + l, :hw] = jnp.ones((l, hw), jnp.float32)

    wl = ewl_ref[pr:pr + 1, :]
    a_ref[pr:pr + 1, :] += wl * a_ref[n - 1:n, :]
    nrm_ref[pr:pr + 1, :hw] = 1.0 + wl[:, :hw] * nrm_ref[n - 1:n, :hw]

    for d in range(K - 1, 0, -1):
        s = 2**d - 1
        sp, m2 = 2 ** (d - 1) - 1, 2 ** (d - 1)
        for o, l in _chunks(m2):
            rp = slice(sp + o, sp + o + l)
            rl = slice(s + 2 * o, s + 2 * o + 2 * l, 2)
            rr = slice(s + 2 * o + 1, s + 2 * o + 2 * l, 2)
            wl = ewl_ref[rp, :]
            wr = ewr_ref[rp, :]
            a_ref[rp, :] += wl * a_ref[rl, :] + wr * a_ref[rr, :]
            nrm_ref[rp, :hw] = (1.0 + wl[:, :hw] * nrm_ref[rl, :hw]
                                + wr[:, :hw] * nrm_ref[rr, :hw])

    for d in range(1, K):
        s = 2**d - 1
        sp, m2 = 2 ** (d - 1) - 1, 2 ** (d - 1)
        for o, l in _chunks(m2):
            rp = slice(sp + o, sp + o + l)
            rl = slice(s + 2 * o, s + 2 * o + 2 * l, 2)
            rr = slice(s + 2 * o + 1, s + 2 * o + 2 * l, 2)
            wl = ewl_ref[rp, :]
            wr = ewr_ref[rp, :]
            p = a_ref[rp, :]
            pn = nrm_ref[rp, :hw]
            al = a_ref[rl, :]
            ar = a_ref[rr, :]
            a_ref[rl, :] = al + wl * (p - wl * al)
            a_ref[rr, :] = ar + wr * (p - wr * ar)
            nl = nrm_ref[rl, :hw]
            nr = nrm_ref[rr, :hw]
            nrm_ref[rl, :hw] = nl + wl[:, :hw] * (pn - wl[:, :hw] * nl)
            nrm_ref[rr, :hw] = nr + wr[:, :hw] * (pn - wr[:, :hw] * nr)
    wl = ewl_ref[pr:pr + 1, :]
    a = a_ref[n - 1:n, :]
    a_ref[n - 1:n, :] = a + wl * (a_ref[pr:pr + 1, :] - wl * a)
    nn = nrm_ref[n - 1:n, :hw]
    nrm_ref[n - 1:n, :hw] = nn + wl[:, :hw] * (nrm_ref[pr:pr + 1, :hw]
                                               - wl[:, :hw] * nn)

    def _flush(buf, sem, bi, q):
        return pltpu.make_async_copy(
            buf, out_hbm.at[bi, :, pl.ds(q * hq, hq), :], sem)

    for q in range(nq):
        buf = osa_ref if q % 2 == 0 else osb_ref
        sem = osema if q % 2 == 0 else osemb
        if q < 2:
            @pl.when(i > 0)
            def _():
                _flush(buf, sem, i - 1, q + nq - 2).wait()
        else:
            _flush(buf, sem, i, q - 2).wait()
        for k in range(0, hq, 8):
            r = slice((q * hq + k) * w, (q * hq + k + 8) * w)
            nv = nrm_ref[r, :hw]
            y = (a_ref[r, :] / jnp.concatenate([nv, nv], axis=1))
            buf[:, k:k + 8, :] = jax.lax.dot_general(
                _eye(c), y.reshape(8, w, c), (((0,), (2,)), ((), ())),
                preferred_element_type=jnp.float32)
        _flush(buf, sem, i, q).start()

    @pl.when(i == b - 1)
    def _():
        for q in (nq - 2, nq - 1):
            _flush(osa_ref if q % 2 == 0 else osb_ref,
                   osema if q % 2 == 0 else osemb, i, q).wait()


def kernel(feature_in, embed_in, tree):
    b, c, h, w = feature_in.shape
    n = h * w
    ce = embed_in.shape[1]
    return pl.pallas_call(
        _tree_dp_kernel,
        grid=(b,),
        in_specs=[
            pl.BlockSpec(memory_space=pltpu.MemorySpace.HBM),
            pl.BlockSpec(memory_space=pltpu.MemorySpace.HBM),
        ],
        out_specs=pl.BlockSpec(memory_space=pltpu.MemorySpace.HBM),
        out_shape=jax.ShapeDtypeStruct((b, c, h, w), jnp.float32),
        scratch_shapes=[
            pltpu.VMEM((n, c), jnp.float32),
            pltpu.VMEM((n, c), jnp.float32),
            pltpu.VMEM((n // 2, c), jnp.float32),
            pltpu.VMEM((n // 2, c), jnp.float32),
            pltpu.VMEM((ce, h, w), jnp.float32),
            pltpu.VMEM((c, h, w), jnp.float32),
            pltpu.VMEM((c, max(h // 4, 8), w), jnp.float32),
            pltpu.VMEM((c, max(h // 4, 8), w), jnp.float32),
            pltpu.SemaphoreType.DMA,
            pltpu.SemaphoreType.DMA,
            pltpu.SemaphoreType.DMA,
            pltpu.SemaphoreType.DMA,
        ],
        compiler_params=pltpu.CompilerParams(
            dimension_semantics=("arbitrary",)),
    )(feature_in, embed_in)
